# baseline (speedup 1.0000x reference)
import dataclasses

import jax
import jax.numpy as jnp
from jax import lax
from jax.experimental import pallas as pl
from jax.experimental.pallas import tpu as pltpu
from jax.experimental.pallas import tpu_sc as plsc

N = 10000
E = 320000
D = 128
NC = 2
NS = 16
NW = NC * NS
CH = 128
KCH = 80
K0 = 104
K1 = 56
NCHUNK = NS * (K0 + K1)
EPAD = NCHUNK * CH
ACC_ROWS = 10240
RPS = ACC_ROWS // NS

_vec_mesh = plsc.VectorSubcoreMesh(core_axis_name="c", subcore_axis_name="s")

_sc_params = pltpu.CompilerParams()
if "needs_layout_passes" in pltpu.CompilerParams.__dataclass_fields__:
    _sc_params = dataclasses.replace(_sc_params, needs_layout_passes=False)


def _hist_body(dst_hbm, out_hbm, dst_v, hist_v):
    cid = lax.axis_index("c")
    sid = lax.axis_index("s")
    wid = sid * NC + cid
    pltpu.sync_copy(dst_hbm.at[wid], dst_v)
    zero16 = jnp.zeros((16,), jnp.float32)
    ones16 = jnp.ones((16,), jnp.float32)

    @pl.loop(0, ACC_ROWS // 16)
    def _zero(r):
        hist_v[r, :] = zero16

    @pl.loop(0, KCH)
    def _chunk(j):
        @pl.loop(0, CH // 16)
        def _grp(c):
            idx = dst_v[j, pl.ds(c * 16, 16)]
            row = lax.shift_right_logical(idx, 4)
            col = lax.bitwise_and(idx, 15)
            plsc.addupdate_scatter(hist_v, [row, col], ones16)

    pltpu.sync_copy(hist_v, out_hbm.at[wid])


def _sc_hist(dst3):
    k = pl.kernel(
        _hist_body,
        out_type=jax.ShapeDtypeStruct((NW, ACC_ROWS // 16, 16), jnp.float32),
        mesh=_vec_mesh,
        scratch_types=[
            pltpu.VMEM((KCH, CH), jnp.int32),
            pltpu.VMEM((ACC_ROWS // 16, 16), jnp.float32),
        ],
        compiler_params=_sc_params,
    )
    return k(dst3)


def _agg_branch(K, base, sid, g_hbm, src_hbm, dst_hbm, out_wb,
                src_v, dst_v, buf, acc_sh):
    sv = src_v.at[pl.ds(0, K)]
    dv = dst_v.at[pl.ds(0, K)]
    pltpu.sync_copy(src_hbm.at[pl.ds(base, K)], sv)
    pltpu.sync_copy(dst_hbm.at[pl.ds(base, K)], dv)
    plsc.subcore_barrier()

    @pl.loop(0, K)
    def _edge_chunk(j):
        pltpu.sync_copy(g_hbm.at[sv.at[j]], buf)
        pltpu.sync_copy(buf, acc_sh.at[dv.at[j]], add=True)

    plsc.subcore_barrier()
    pltpu.sync_copy(
        acc_sh.at[pl.ds(sid * RPS, RPS)],
        out_wb.at[pl.ds(sid * RPS, RPS)],
    )


def _agg_body(g_hbm, src_hbm, dst_hbm, out_hbm, src_v, dst_v, buf, acc_sh):
    cid = lax.axis_index("c")
    sid = lax.axis_index("s")
    zero16 = jnp.zeros((16,), jnp.float32)

    @pl.loop(0, CH)
    def _zr(r):
        @pl.loop(0, D // 16)
        def _zc(c):
            buf[r, pl.ds(c * 16, 16)] = zero16

    @pl.loop(0, RPS // CH)
    def _zacc(k):
        pltpu.sync_copy(buf, acc_sh.at[pl.ds(sid * RPS + k * CH, CH)])

    @pl.when(cid == 0)
    def _c0():
        _agg_branch(K0, sid * K0, sid, g_hbm, src_hbm, dst_hbm,
                    out_hbm.at[0], src_v, dst_v, buf, acc_sh)

    @pl.when(cid == 1)
    def _c1():
        _agg_branch(K1, NS * K0 + sid * K1, sid, g_hbm, src_hbm, dst_hbm,
                    out_hbm.at[1], src_v, dst_v, buf, acc_sh)


def _sc_aggregate(g, src2c, dst2c):
    k = pl.kernel(
        _agg_body,
        out_type=jax.ShapeDtypeStruct((NC, ACC_ROWS, D), jnp.float32),
        mesh=_vec_mesh,
        scratch_types=[
            pltpu.VMEM((K0, CH), jnp.int32),
            pltpu.VMEM((K0, CH), jnp.int32),
            pltpu.VMEM((CH, D), jnp.float32),
            pltpu.VMEM_SHARED((ACC_ROWS, D), jnp.float32),
        ],
        compiler_params=_sc_params,
    )
    return k(g, src2c, dst2c)


_RB = 2000


def _dinv_body(h_ref, o_ref):
    deg = jnp.sum(h_ref[...], axis=0) + 1.0
    o_ref[...] = 1.0 / jnp.sqrt(deg)


def _tc_dinv(hist4):
    return pl.pallas_call(
        _dinv_body,
        out_shape=jax.ShapeDtypeStruct((ACC_ROWS // D, D), jnp.float32),
    )(hist4)


def _mm_scale_body(x_ref, w_ref, dinv_ref, o_ref):
    h = lax.dot_general(
        x_ref[...], w_ref[...], (((1,), (0,)), ((), ())),
        precision=lax.Precision.HIGHEST,
    )
    o_ref[...] = h * dinv_ref[...]


def _tc_mm_scale(x, W, dinv):
    full = pl.BlockSpec((_RB, D), lambda i: (i, 0))
    col = pl.BlockSpec((_RB, 1), lambda i: (i, 0))
    return pl.pallas_call(
        _mm_scale_body,
        grid=(N // _RB,),
        in_specs=[full, pl.BlockSpec((D, D), lambda i: (0, 0)), col],
        out_specs=full,
        out_shape=jax.ShapeDtypeStruct((N, D), jnp.float32),
    )(x, W, dinv)


def _layer2_body(p0_ref, p1_ref, g1_ref, dinv_ref, b1_ref, w2_ref, o_ref):
    u = (p0_ref[...] + p1_ref[...] + g1_ref[...]) * dinv_ref[...] + b1_ref[...]
    h = jnp.maximum(u, 0.0)
    o_ref[...] = lax.dot_general(
        h, w2_ref[...], (((1,), (0,)), ((), ())),
        precision=lax.Precision.HIGHEST,
    ) * dinv_ref[...]


def _tc_layer2(p0, p1, g1, dinv, b1r, W2):
    full = pl.BlockSpec((_RB, D), lambda i: (i, 0))
    col = pl.BlockSpec((_RB, 1), lambda i: (i, 0))
    return pl.pallas_call(
        _layer2_body,
        grid=(N // _RB,),
        in_specs=[full, full, full, col,
                  pl.BlockSpec((1, D), lambda i: (0, 0)),
                  pl.BlockSpec((D, D), lambda i: (0, 0))],
        out_specs=full,
        out_shape=jax.ShapeDtypeStruct((N, D), jnp.float32),
    )(p0, p1, g1, dinv, b1r, W2)


def _final_body(q0_ref, q1_ref, g2_ref, dinv_ref, b2_ref, o_ref):
    o_ref[...] = (
        (q0_ref[...] + q1_ref[...] + g2_ref[...]) * dinv_ref[...] + b2_ref[...]
    )


def _tc_final(q0, q1, g2, dinv, b2r):
    full = pl.BlockSpec((_RB, D), lambda i: (i, 0))
    col = pl.BlockSpec((_RB, 1), lambda i: (i, 0))
    return pl.pallas_call(
        _final_body,
        grid=(N // _RB,),
        in_specs=[full, full, full, col,
                  pl.BlockSpec((1, D), lambda i: (0, 0))],
        out_specs=full,
        out_shape=jax.ShapeDtypeStruct((N, D), jnp.float32),
    )(q0, q1, g2, dinv, b2r)


def kernel(x, edge_index, W1, b1, W2, b2):
    ei = edge_index.astype(jnp.int32)
    pad = EPAD - E
    src = jnp.concatenate([ei[0], jnp.zeros((pad,), jnp.int32)])
    dump = N + (jnp.arange(pad, dtype=jnp.int32) % (ACC_ROWS - N))
    dst = jnp.concatenate([ei[1], dump])
    src2c = src.reshape(NCHUNK, CH)
    dst2c = dst.reshape(NCHUNK, CH)
    dst3 = dst.reshape(NW, KCH, CH)

    hist = _sc_hist(dst3)
    hist4 = hist.reshape(NW, ACC_ROWS // D, D)
    dinv = _tc_dinv(hist4).reshape(ACC_ROWS, 1)[:N]

    g1 = _tc_mm_scale(x, W1, dinv)
    p = _sc_aggregate(g1, src2c, dst2c)
    g2 = _tc_layer2(p[0, :N], p[1, :N], g1, dinv, b1.reshape(1, D), W2)
    q = _sc_aggregate(g2, src2c, dst2c)
    return _tc_final(q[0, :N], q[1, :N], g2, dinv, b2.reshape(1, D))


# speedup vs baseline: 1.3969x; 1.3969x over previous
import dataclasses

import jax
import jax.numpy as jnp
from jax import lax
from jax.experimental import pallas as pl
from jax.experimental.pallas import tpu as pltpu
from jax.experimental.pallas import tpu_sc as plsc

N = 10000
E = 320000
D = 128
NC = 2
NS = 16
NW = NC * NS
CH = 128
KCH = 79
EPAD = NW * KCH * CH
ACC_ROWS = 10240
RPS = ACC_ROWS // NS

_vec_mesh = plsc.VectorSubcoreMesh(core_axis_name="c", subcore_axis_name="s")

_sc_params = pltpu.CompilerParams()
if "needs_layout_passes" in pltpu.CompilerParams.__dataclass_fields__:
    _sc_params = dataclasses.replace(_sc_params, needs_layout_passes=False)


def _hist_body(dst_hbm, out_hbm, dst_v, hist_v):
    cid = lax.axis_index("c")
    sid = lax.axis_index("s")
    wid = sid * NC + cid
    pltpu.sync_copy(dst_hbm.at[wid], dst_v)
    zero16 = jnp.zeros((16,), jnp.float32)
    ones16 = jnp.ones((16,), jnp.float32)

    @pl.loop(0, ACC_ROWS // 16)
    def _zero(r):
        hist_v[r, :] = zero16

    @pl.loop(0, KCH)
    def _chunk(j):
        @pl.loop(0, CH // 16)
        def _grp(c):
            idx = dst_v[j, pl.ds(c * 16, 16)]
            row = lax.shift_right_logical(idx, 4)
            col = lax.bitwise_and(idx, 15)
            plsc.addupdate_scatter(hist_v, [row, col], ones16)

    pltpu.sync_copy(hist_v, out_hbm.at[wid])


def _sc_hist(dst3):
    k = pl.kernel(
        _hist_body,
        out_type=jax.ShapeDtypeStruct((NW, ACC_ROWS // 16, 16), jnp.float32),
        mesh=_vec_mesh,
        scratch_types=[
            pltpu.VMEM((KCH, CH), jnp.int32),
            pltpu.VMEM((ACC_ROWS // 16, 16), jnp.float32),
        ],
        compiler_params=_sc_params,
    )
    return k(dst3)


def _agg_body(g_hbm, src_hbm, dst_hbm, out_hbm, src_v, dst_v, buf, acc_sh):
    cid = lax.axis_index("c")
    sid = lax.axis_index("s")
    wid = sid * NC + cid
    zero16 = jnp.zeros((16,), jnp.float32)

    @pl.loop(0, CH)
    def _zr(r):
        @pl.loop(0, D // 16)
        def _zc(c):
            buf[r, pl.ds(c * 16, 16)] = zero16

    @pl.loop(0, RPS // CH)
    def _zacc(k):
        pltpu.sync_copy(buf, acc_sh.at[pl.ds(sid * RPS + k * CH, CH)])

    pltpu.sync_copy(src_hbm.at[wid], src_v)
    pltpu.sync_copy(dst_hbm.at[wid], dst_v)
    plsc.subcore_barrier()

    @pl.loop(0, KCH)
    def _edge_chunk(j):
        pltpu.sync_copy(g_hbm.at[src_v.at[j]], buf)
        pltpu.sync_copy(buf, acc_sh.at[dst_v.at[j]], add=True)

    plsc.subcore_barrier()
    pltpu.sync_copy(
        acc_sh.at[pl.ds(sid * RPS, RPS)],
        out_hbm.at[cid].at[pl.ds(sid * RPS, RPS)],
    )


def _sc_aggregate(g, src3, dst3):
    k = pl.kernel(
        _agg_body,
        out_type=jax.ShapeDtypeStruct((NC, ACC_ROWS, D), jnp.float32),
        mesh=_vec_mesh,
        scratch_types=[
            pltpu.VMEM((KCH, CH), jnp.int32),
            pltpu.VMEM((KCH, CH), jnp.int32),
            pltpu.VMEM((CH, D), jnp.float32),
            pltpu.VMEM_SHARED((ACC_ROWS, D), jnp.float32),
        ],
        compiler_params=_sc_params,
    )
    return k(g, src3, dst3)


_RB = 2000


def _dinv_body(h_ref, o_ref):
    deg = jnp.sum(h_ref[...], axis=0) + 1.0
    o_ref[...] = 1.0 / jnp.sqrt(deg)


def _tc_dinv(hist4):
    return pl.pallas_call(
        _dinv_body,
        out_shape=jax.ShapeDtypeStruct((ACC_ROWS // D, D), jnp.float32),
    )(hist4)


def _mm_scale_body(x_ref, w_ref, dinv_ref, o_ref):
    h = lax.dot_general(
        x_ref[...], w_ref[...], (((1,), (0,)), ((), ())),
        precision=lax.Precision.HIGHEST,
    )
    o_ref[...] = h * dinv_ref[...]


def _tc_mm_scale(x, W, dinv):
    full = pl.BlockSpec((_RB, D), lambda i: (i, 0))
    col = pl.BlockSpec((_RB, 1), lambda i: (i, 0))
    return pl.pallas_call(
        _mm_scale_body,
        grid=(N // _RB,),
        in_specs=[full, pl.BlockSpec((D, D), lambda i: (0, 0)), col],
        out_specs=full,
        out_shape=jax.ShapeDtypeStruct((N, D), jnp.float32),
    )(x, W, dinv)


def _layer2_body(p0_ref, p1_ref, g1_ref, dinv_ref, b1_ref, w2_ref, o_ref):
    u = (p0_ref[...] + p1_ref[...] + g1_ref[...]) * dinv_ref[...] + b1_ref[...]
    h = jnp.maximum(u, 0.0)
    o_ref[...] = lax.dot_general(
        h, w2_ref[...], (((1,), (0,)), ((), ())),
        precision=lax.Precision.HIGHEST,
    ) * dinv_ref[...]


def _tc_layer2(p0, p1, g1, dinv, b1r, W2):
    full = pl.BlockSpec((_RB, D), lambda i: (i, 0))
    col = pl.BlockSpec((_RB, 1), lambda i: (i, 0))
    return pl.pallas_call(
        _layer2_body,
        grid=(N // _RB,),
        in_specs=[full, full, full, col,
                  pl.BlockSpec((1, D), lambda i: (0, 0)),
                  pl.BlockSpec((D, D), lambda i: (0, 0))],
        out_specs=full,
        out_shape=jax.ShapeDtypeStruct((N, D), jnp.float32),
    )(p0, p1, g1, dinv, b1r, W2)


def _final_body(q0_ref, q1_ref, g2_ref, dinv_ref, b2_ref, o_ref):
    o_ref[...] = (
        (q0_ref[...] + q1_ref[...] + g2_ref[...]) * dinv_ref[...] + b2_ref[...]
    )


def _tc_final(q0, q1, g2, dinv, b2r):
    full = pl.BlockSpec((_RB, D), lambda i: (i, 0))
    col = pl.BlockSpec((_RB, 1), lambda i: (i, 0))
    return pl.pallas_call(
        _final_body,
        grid=(N // _RB,),
        in_specs=[full, full, full, col,
                  pl.BlockSpec((1, D), lambda i: (0, 0))],
        out_specs=full,
        out_shape=jax.ShapeDtypeStruct((N, D), jnp.float32),
    )(q0, q1, g2, dinv, b2r)


def kernel(x, edge_index, W1, b1, W2, b2):
    ei = edge_index.astype(jnp.int32)
    pad = EPAD - E
    src = jnp.concatenate([ei[0], jnp.zeros((pad,), jnp.int32)])
    dump = N + (jnp.arange(pad, dtype=jnp.int32) % (ACC_ROWS - N))
    dst = jnp.concatenate([ei[1], dump])
    src3 = src.reshape(NW, KCH, CH)
    dst3 = dst.reshape(NW, KCH, CH)

    hist = _sc_hist(dst3)
    hist4 = hist.reshape(NW, ACC_ROWS // D, D)
    dinv = _tc_dinv(hist4).reshape(ACC_ROWS, 1)[:N]

    g1 = _tc_mm_scale(x, W1, dinv)
    p = _sc_aggregate(g1, src3, dst3)
    g2 = _tc_layer2(p[0, :N], p[1, :N], g1, dinv, b1.reshape(1, D), W2)
    q = _sc_aggregate(g2, src3, dst3)
    return _tc_final(q[0, :N], q[1, :N], g2, dinv, b2.reshape(1, D))
